# baseline (speedup 1.0000x reference)
import functools

import jax
import jax.numpy as jnp
from jax import lax
from jax.experimental import pallas as pl
from jax.experimental.pallas import tpu as pltpu
from jax.experimental.pallas import tpu_sc as plsc

NS = 16
L = 16


@functools.lru_cache(maxsize=None)
def _make_sc(N, V, B, T):
    RPW = N // NS
    VB = V // 128
    mesh = plsc.VectorSubcoreMesh(
        core_axis_name="c", subcore_axis_name="s", num_cores=1)

    @functools.partial(
        pl.kernel,
        out_type=jax.ShapeDtypeStruct((L,), jnp.float32),
        mesh=mesh,
        compiler_params=pltpu.CompilerParams(
            needs_layout_passes=False,
            skip_device_barrier=True,
            disable_bounds_checks=True,
        ),
        scratch_types=[
            pltpu.VMEM((RPW,), jnp.int32),
            pltpu.VMEM((RPW,), jnp.float32),
            pltpu.VMEM((RPW,), jnp.int32),
            pltpu.VMEM((RPW, 128), jnp.float32),
            pltpu.VMEM((128,), jnp.float32),
            pltpu.VMEM((NS, 128), jnp.float32),
            pltpu.VMEM((L,), jnp.float32),
            pltpu.VMEM_SHARED((NS, 128), jnp.float32),
            pltpu.SemaphoreType.DMA,
            pltpu.SemaphoreType.DMA,
            pltpu.SemaphoreType.DMA,
            pltpu.SemaphoreType.DMA,
            pltpu.SemaphoreType.DMA,
            pltpu.SemaphoreType.DMA,
        ],
    )
    def k(in_hbm, tgt_hbm, msk_hbm, out_hbm,
          tgt_v, msk_v, idx_v, rows_v, acc_v, all_v, o_v, shared,
          sem, sem2, g0, g1, g2, g3):
        gsems = [g0, g1, g2, g3]
        sid = lax.axis_index("s")
        base = sid * RPW
        b = base // T
        blk = (base % T) // 128
        NB = RPW // 128
        NCH = 4
        CH = RPW // NCH
        JCH = CH // L
        flat = in_hbm.reshape(N * V // 128, 128)
        for i in range(NB):
            pltpu.async_copy(tgt_hbm.at[blk + i, b],
                             tgt_v.at[pl.ds(i * 128, 128)], sem)
            pltpu.async_copy(msk_hbm.at[blk + i, b],
                             msk_v.at[pl.ds(i * 128, 128)], sem2)
        for i in range(NB):
            pltpu.make_async_copy(tgt_hbm.at[blk, b],
                                  tgt_v.at[pl.ds(0, 128)], sem).wait()
        lane = lax.iota(jnp.int32, L)
        gathers = []
        for c in range(NCH):
            for jj in range(JCH):
                j = c * JCH + jj
                t = tgt_v[pl.ds(j * L, L)]
                n = (base + j * L) + lane
                q = (lax.shift_right_logical(n, 3) * (VB * 8)
                     + lax.shift_right_logical(t, 7) * 8
                     + jnp.bitwise_and(n, 7))
                idx_v[pl.ds(j * L, L)] = q
            gathers.append(pltpu.async_copy(
                flat.at[idx_v.at[pl.ds(c * CH, CH)]],
                rows_v.at[pl.ds(c * CH, CH)], gsems[c]))
        for i in range(NB):
            pltpu.make_async_copy(msk_hbm.at[blk, b],
                                  msk_v.at[pl.ds(0, 128)], sem2).wait()
        acc = jnp.zeros((L,), jnp.float32)
        cnt = jnp.zeros((L,), jnp.float32)
        for c in range(NCH):
            gathers[c].wait()
            for jj in range(JCH):
                j = c * JCH + jj
                t = tgt_v[pl.ds(j * L, L)]
                v = plsc.load_gather(
                    rows_v, [j * L + lane, jnp.bitwise_and(t, 127)])
                m = msk_v[pl.ds(j * L, L)]
                acc = acc - v * m
                cnt = cnt + m
        acc_v[pl.ds(0, L)] = acc
        acc_v[pl.ds(L, L)] = cnt
        pltpu.sync_copy(acc_v, shared.at[sid])
        plsc.subcore_barrier()

        @pl.when(sid == 0)
        def _():
            pltpu.sync_copy(shared, all_v)
            s = jnp.zeros((L,), jnp.float32)
            c = jnp.zeros((L,), jnp.float32)
            for i in range(NS):
                s = s + all_v[i, pl.ds(0, L)]
                c = c + all_v[i, pl.ds(L, L)]
            S = lax.broadcast_in_dim(
                lax.reduce_sum_p.bind(s, axes=(0,)), (L,), ())
            C = lax.broadcast_in_dim(
                lax.reduce_sum_p.bind(c, axes=(0,)), (L,), ())
            o_v[...] = S / C
            pltpu.sync_copy(o_v, out_hbm)

    return k


def kernel(input, target, mask):
    B, T, V = input.shape
    target = target[:, :T]
    mask = mask[:, :T]
    N = B * T
    x5 = input.reshape(B, T // 8, 8, V // 128, 128).transpose(0, 1, 3, 2, 4)
    tgt = target.astype(jnp.int32).reshape(B, T // 128, 128).transpose(1, 0, 2)
    msk = mask.astype(jnp.float32).reshape(B, T // 128, 128).transpose(1, 0, 2)
    out = _make_sc(N, V, B, T)(x5, tgt, msk)
    return out[0]


# speedup vs baseline: 1.0055x; 1.0055x over previous
import functools

import jax
import jax.numpy as jnp
from jax import lax
from jax.experimental import pallas as pl
from jax.experimental.pallas import tpu as pltpu
from jax.experimental.pallas import tpu_sc as plsc

NS = 16
L = 16


@functools.lru_cache(maxsize=None)
def _make_sc(N, V, B, T):
    RPW = N // NS
    VB = V // 128
    mesh = plsc.VectorSubcoreMesh(
        core_axis_name="c", subcore_axis_name="s", num_cores=1)

    @functools.partial(
        pl.kernel,
        out_type=jax.ShapeDtypeStruct((L,), jnp.float32),
        mesh=mesh,
        compiler_params=pltpu.CompilerParams(
            needs_layout_passes=False,
            skip_device_barrier=True,
            disable_bounds_checks=True,
        ),
        scratch_types=[
            pltpu.VMEM((RPW,), jnp.int32),
            pltpu.VMEM((RPW,), jnp.float32),
            pltpu.VMEM((RPW,), jnp.int32),
            pltpu.VMEM((RPW, 128), jnp.float32),
            pltpu.VMEM((128,), jnp.float32),
            pltpu.VMEM((NS, 128), jnp.float32),
            pltpu.VMEM((L,), jnp.float32),
            pltpu.VMEM_SHARED((NS, 128), jnp.float32),
            pltpu.SemaphoreType.DMA,
            pltpu.SemaphoreType.DMA,
            pltpu.SemaphoreType.DMA,
            pltpu.SemaphoreType.DMA,
            pltpu.SemaphoreType.DMA,
            pltpu.SemaphoreType.DMA,
        ],
    )
    def k(in_hbm, tgt_hbm, msk_hbm, out_hbm,
          tgt_v, msk_v, idx_v, rows_v, acc_v, all_v, o_v, shared,
          sem, sem2, g0, g1, g2, g3):
        gsems = [g0, g1, g2, g3]
        sid = lax.axis_index("s")
        base = sid * RPW
        b = base // T
        blk = (base % T) // 128
        NB = RPW // 128
        NCH = 2
        CH = RPW // NCH
        JCH = CH // L
        flat = in_hbm.reshape(N * V // 128, 128)
        for i in range(NB):
            pltpu.async_copy(tgt_hbm.at[blk + i, b],
                             tgt_v.at[pl.ds(i * 128, 128)], sem)
            pltpu.async_copy(msk_hbm.at[blk + i, b],
                             msk_v.at[pl.ds(i * 128, 128)], sem2)
        for i in range(NB):
            pltpu.make_async_copy(tgt_hbm.at[blk, b],
                                  tgt_v.at[pl.ds(0, 128)], sem).wait()
        lane = lax.iota(jnp.int32, L)

        def idx_body(j, _):
            t = tgt_v[pl.ds(pl.multiple_of(j * L, L), L)]
            n = (base + j * L) + lane
            q = (lax.shift_right_logical(n, 3) * (VB * 8)
                 + lax.shift_right_logical(t, 7) * 8
                 + jnp.bitwise_and(n, 7))
            idx_v[pl.ds(pl.multiple_of(j * L, L), L)] = q
            return 0

        gathers = []
        for c in range(NCH):
            lax.fori_loop(c * JCH, (c + 1) * JCH, idx_body, 0)
            gathers.append(pltpu.async_copy(
                flat.at[idx_v.at[pl.ds(c * CH, CH)]],
                rows_v.at[pl.ds(c * CH, CH)], gsems[c]))
        for i in range(NB):
            pltpu.make_async_copy(msk_hbm.at[blk, b],
                                  msk_v.at[pl.ds(0, 128)], sem2).wait()

        def sel_body(j, carry):
            acc, cnt = carry
            t = tgt_v[pl.ds(pl.multiple_of(j * L, L), L)]
            v = plsc.load_gather(
                rows_v, [j * L + lane, jnp.bitwise_and(t, 127)])
            m = msk_v[pl.ds(pl.multiple_of(j * L, L), L)]
            return acc - v * m, cnt + m

        acc = jnp.zeros((L,), jnp.float32)
        cnt = jnp.zeros((L,), jnp.float32)
        for c in range(NCH):
            gathers[c].wait()
            acc, cnt = lax.fori_loop(
                c * JCH, (c + 1) * JCH, sel_body, (acc, cnt))
        acc_v[pl.ds(0, L)] = acc
        acc_v[pl.ds(L, L)] = cnt
        pltpu.sync_copy(acc_v, shared.at[sid])
        plsc.subcore_barrier()

        @pl.when(sid == 0)
        def _():
            pltpu.sync_copy(shared, all_v)

            def red_body(i, carry):
                s, c = carry
                return (s + all_v[i, pl.ds(0, L)],
                        c + all_v[i, pl.ds(L, L)])

            s, c = lax.fori_loop(
                0, NS, red_body,
                (jnp.zeros((L,), jnp.float32), jnp.zeros((L,), jnp.float32)))
            S = lax.broadcast_in_dim(
                lax.reduce_sum_p.bind(s, axes=(0,)), (L,), ())
            C = lax.broadcast_in_dim(
                lax.reduce_sum_p.bind(c, axes=(0,)), (L,), ())
            o_v[...] = S / C
            pltpu.sync_copy(o_v, out_hbm)

    return k


def kernel(input, target, mask):
    B, T, V = input.shape
    target = target[:, :T]
    mask = mask[:, :T]
    N = B * T
    x5 = input.reshape(B, T // 8, 8, V // 128, 128).transpose(0, 1, 3, 2, 4)
    tgt = target.astype(jnp.int32).reshape(B, T // 128, 128).transpose(1, 0, 2)
    msk = mask.astype(jnp.float32).reshape(B, T // 128, 128).transpose(1, 0, 2)
    out = _make_sc(N, V, B, T)(x5, tgt, msk)
    return out[0]
